# baseline (speedup 1.0000x reference)
import jax
import jax.numpy as jnp
from jax.experimental import pallas as pl
from jax.experimental.pallas import tpu as pltpu

_P = 0.1
_NBUF = 4
_CBM = 4
_XCB = 8


def _body(rand_ref, mask_hbm, x_hbm, out_hbm, xout_hbm,
          keep_buf, mask_buf, res_buf, sem_in, sem_out, sem_x):
    C = mask_hbm.shape[0]
    nx = C // _XCB
    nsteps = C // _CBM

    for j in range(nx):
        pltpu.make_async_copy(
            x_hbm.at[pl.ds(j * _XCB, _XCB)],
            xout_hbm.at[pl.ds(j * _XCB, _XCB)],
            sem_x.at[j],
        ).start()

    keep_buf[...] = jnp.broadcast_to(
        (rand_ref[...] >= _P).astype(jnp.float32), keep_buf.shape)

    def in_copy(step, slot):
        return pltpu.make_async_copy(
            mask_hbm.at[pl.ds(step * _CBM, _CBM)],
            mask_buf.at[slot],
            sem_in.at[slot],
        )

    def out_copy(step, slot):
        return pltpu.make_async_copy(
            res_buf.at[slot],
            out_hbm.at[pl.ds(step * _CBM, _CBM)],
            sem_out.at[slot],
        )

    for slot in range(_NBUF):
        in_copy(slot, slot).start()

    for step in range(nsteps):
        slot = step % _NBUF
        in_copy(step, slot).wait()
        if step >= _NBUF:
            out_copy(step - _NBUF, slot).wait()
        res_buf[slot] = mask_buf[slot] * keep_buf[...][None]
        out_copy(step, slot).start()
        nxt = step + _NBUF
        if nxt < nsteps:
            in_copy(nxt, slot).start()

    for step in range(nsteps - _NBUF, nsteps):
        out_copy(step, step % _NBUF).wait()
    for j in range(nx):
        pltpu.make_async_copy(
            x_hbm.at[pl.ds(j * _XCB, _XCB)],
            xout_hbm.at[pl.ds(j * _XCB, _XCB)],
            sem_x.at[j],
        ).wait()


def kernel(x, mask, rand_t):
    C, T, V = mask.shape
    out, x_out = pl.pallas_call(
        _body,
        in_specs=[
            pl.BlockSpec(memory_space=pltpu.VMEM),
            pl.BlockSpec(memory_space=pltpu.MemorySpace.HBM),
            pl.BlockSpec(memory_space=pltpu.MemorySpace.HBM),
        ],
        out_specs=[
            pl.BlockSpec(memory_space=pltpu.MemorySpace.HBM),
            pl.BlockSpec(memory_space=pltpu.MemorySpace.HBM),
        ],
        out_shape=[
            jax.ShapeDtypeStruct((C, T, V), jnp.float32),
            jax.ShapeDtypeStruct((C, T, V), jnp.float32),
        ],
        scratch_shapes=[
            pltpu.VMEM((T, V), jnp.float32),
            pltpu.VMEM((_NBUF, _CBM, T, V), jnp.float32),
            pltpu.VMEM((_NBUF, _CBM, T, V), jnp.float32),
            pltpu.SemaphoreType.DMA((_NBUF,)),
            pltpu.SemaphoreType.DMA((_NBUF,)),
            pltpu.SemaphoreType.DMA((C // _XCB,)),
        ],
    )(rand_t.reshape(T, 1), mask, x)
    return (x_out, out)


# speedup vs baseline: 8.2161x; 8.2161x over previous
import jax
import jax.numpy as jnp
from jax.experimental import pallas as pl

_P = 0.1
_TB = 64


def _expand_body(rand_ref, keep_ref):
    keep = (rand_ref[...] >= _P).astype(jnp.float32)
    keep_ref[...] = jnp.broadcast_to(keep, keep_ref.shape)


def _mul_body(keep_ref, mask_ref, x_ref, out_ref, xout_ref):
    out_ref[...] = mask_ref[...] * keep_ref[...][None]
    xout_ref[...] = x_ref[...]


def kernel(x, mask, rand_t):
    C, T, V = mask.shape

    keep_tv = pl.pallas_call(
        _expand_body,
        out_shape=jax.ShapeDtypeStruct((T, V), jnp.float32),
    )(rand_t.reshape(T, 1))

    blk = pl.BlockSpec((C, _TB, V), lambda i: (0, i, 0))
    out, x_out = pl.pallas_call(
        _mul_body,
        grid=(T // _TB,),
        in_specs=[
            pl.BlockSpec((_TB, V), lambda i: (i, 0)),
            blk,
            blk,
        ],
        out_specs=[blk, blk],
        out_shape=[
            jax.ShapeDtypeStruct((C, T, V), jnp.float32),
            jax.ShapeDtypeStruct((C, T, V), jnp.float32),
        ],
    )(keep_tv, mask, x)
    return (x_out, out)


# speedup vs baseline: 52.1014x; 6.3414x over previous
import jax
import jax.numpy as jnp
from jax.experimental import pallas as pl

_P = 0.1
_CB = 8


def _body(rand_ref, mask_ref, x_ref, out_ref, xout_ref):
    keep = (rand_ref[...] >= _P).astype(jnp.float32)
    out_ref[...] = mask_ref[...] * keep[None]
    xout_ref[...] = x_ref[...]


def kernel(x, mask, rand_t):
    C, T, V = mask.shape
    mask_t = jnp.transpose(mask, (0, 2, 1))
    x_t = jnp.transpose(x, (0, 2, 1))

    blk = pl.BlockSpec((_CB, V, T), lambda i: (i, 0, 0))
    out_t, xout_t = pl.pallas_call(
        _body,
        grid=(C // _CB,),
        in_specs=[
            pl.BlockSpec((1, T), lambda i: (0, 0)),
            blk,
            blk,
        ],
        out_specs=[blk, blk],
        out_shape=[
            jax.ShapeDtypeStruct((C, V, T), jnp.float32),
            jax.ShapeDtypeStruct((C, V, T), jnp.float32),
        ],
    )(rand_t.reshape(1, T), mask_t, x_t)
    return (jnp.transpose(xout_t, (0, 2, 1)), jnp.transpose(out_t, (0, 2, 1)))
